# baseline (speedup 1.0000x reference)
import functools

import jax
import jax.numpy as jnp
from jax import lax
from jax.experimental import pallas as pl
from jax.experimental.pallas import tpu as pltpu
from jax.experimental.pallas import tpu_sc as plsc

N = 10000
E = 320000
IN_DIM = 128
DIM = 64
EDIM = 16

NPAD = 10240
EPAD = 327680
NW = 32
EPW = EPAD // NW
WWIN = 1024
NWIN = EPW // WWIN

NBLK = 2048
NGRID = NPAD // NBLK
EBLK = 8192
EGRID = (E + EBLK - 1) // EBLK

_NEG = -1e30


def _celu(v):
    return jnp.where(v > 0, v, jnp.exp(jnp.minimum(v, 0.0)) - 1.0)


def _edge_const_kernel(attrT_ref, we_ref, wte_ref, ae_ref, mx_ref, mxs):
    i = pl.program_id(0)
    wvec = jnp.dot(we_ref[...], wte_ref[...], preferred_element_type=jnp.float32)
    a = attrT_ref[...]
    col = i * EBLK + lax.broadcasted_iota(jnp.int32, (EDIM, EBLK), 1)
    prod = jnp.where(col < E, a * wvec[:, None], 0.0)
    ae = jnp.sum(prod, axis=0)
    ae_ref[...] = ae

    @pl.when(i == 0)
    def _():
        mxs[0] = jnp.float32(_NEG)

    aemask = jnp.where(col[0] < E, ae, _NEG)
    mxs[0] = jnp.maximum(mxs[0], jnp.max(aemask))
    mx_ref[...] = jnp.full((8, 128), mxs[0], jnp.float32)


def _edge_const(attrT, We, wta_e):
    return pl.pallas_call(
        _edge_const_kernel,
        grid=(EGRID,),
        in_specs=[
            pl.BlockSpec((EDIM, EBLK), lambda i: (0, i)),
            pl.BlockSpec((EDIM, DIM), lambda i: (0, 0)),
            pl.BlockSpec((DIM,), lambda i: (0,)),
        ],
        out_specs=[
            pl.BlockSpec((EBLK,), lambda i: (i,)),
            pl.BlockSpec((8, 128), lambda i: (0, 0)),
        ],
        out_shape=[
            jax.ShapeDtypeStruct((E,), jnp.float32),
            jax.ShapeDtypeStruct((8, 128), jnp.float32),
        ],
        scratch_shapes=[pltpu.SMEM((1,), jnp.float32)],
    )(attrT, We, wta_e)


def _pre_kernel(x_ref, w0_ref, b0_ref, wn_ref, wi_ref, wj_ref, we_ref, ws_ref,
                h0_ref, ai_ref, aj_ref, st_ref, wews_ref, sm):
    i = pl.program_id(0)
    h0 = _celu(jnp.dot(x_ref[...], w0_ref[...],
                       preferred_element_type=jnp.float32) + b0_ref[...][None, :])
    h0_ref[...] = h0
    xp = jnp.dot(h0, wn_ref[...], preferred_element_type=jnp.float32)
    ai = jnp.sum(xp * wi_ref[...][None, :], axis=1)
    aj = jnp.sum(xp * wj_ref[...][None, :], axis=1)
    ai_ref[...] = ai
    aj_ref[...] = aj

    @pl.when(i == 0)
    def _():
        sm[0] = jnp.float32(_NEG)
        sm[1] = jnp.float32(_NEG)
        wews_ref[...] = jnp.dot(we_ref[...], ws_ref[...],
                                preferred_element_type=jnp.float32)

    sm[0] = jnp.maximum(sm[0], jnp.max(ai))
    sm[1] = jnp.maximum(sm[1], jnp.max(aj))

    @pl.when(i == NGRID - 1)
    def _():
        row = lax.broadcasted_iota(jnp.int32, (8, 128), 0)
        st_ref[...] = jnp.where(row == 0, sm[0], sm[1])


def _prologue(x_p, W0, b0, Wn, wta_i, wta_j, We, Ws):
    return pl.pallas_call(
        _pre_kernel,
        grid=(NGRID,),
        in_specs=[
            pl.BlockSpec((NBLK, IN_DIM), lambda i: (i, 0)),
            pl.BlockSpec((IN_DIM, DIM), lambda i: (0, 0)),
            pl.BlockSpec((DIM,), lambda i: (0,)),
            pl.BlockSpec((DIM, DIM), lambda i: (0, 0)),
            pl.BlockSpec((DIM,), lambda i: (0,)),
            pl.BlockSpec((DIM,), lambda i: (0,)),
            pl.BlockSpec((EDIM, DIM), lambda i: (0, 0)),
            pl.BlockSpec((DIM, DIM), lambda i: (0, 0)),
        ],
        out_specs=[
            pl.BlockSpec((NBLK, DIM), lambda i: (i, 0)),
            pl.BlockSpec((NBLK,), lambda i: (i,)),
            pl.BlockSpec((NBLK,), lambda i: (i,)),
            pl.BlockSpec((8, 128), lambda i: (0, 0)),
            pl.BlockSpec((EDIM, DIM), lambda i: (0, 0)),
        ],
        out_shape=[
            jax.ShapeDtypeStruct((NPAD, DIM), jnp.float32),
            jax.ShapeDtypeStruct((NPAD,), jnp.float32),
            jax.ShapeDtypeStruct((NPAD,), jnp.float32),
            jax.ShapeDtypeStruct((8, 128), jnp.float32),
            jax.ShapeDtypeStruct((EDIM, DIM), jnp.float32),
        ],
        scratch_shapes=[pltpu.SMEM((2,), jnp.float32)],
    )(x_p, W0, b0, Wn, wta_i, wta_j, We, Ws)


def _sc_round(ai, aj, ae, src, dst3, attr_p, sv):
    mesh = plsc.VectorSubcoreMesh(core_axis_name="c", subcore_axis_name="s")

    @functools.partial(
        pl.kernel,
        out_type=[
            jax.ShapeDtypeStruct((2, NPAD, EDIM), jnp.float32),
            jax.ShapeDtypeStruct((2, NPAD), jnp.float32),
        ],
        mesh=mesh,
        compiler_params=pltpu.CompilerParams(needs_layout_passes=False,
                                             use_tc_tiling_on_sc=False),
        scratch_types=[
            pltpu.VMEM((NPAD,), jnp.float32),
            pltpu.VMEM((NPAD,), jnp.float32),
            pltpu.VMEM((EPW,), jnp.float32),
            pltpu.VMEM((EPW,), jnp.int32),
            pltpu.VMEM((EPW // 128, 128), jnp.int32),
            pltpu.VMEM((2, WWIN, EDIM), jnp.float32),
            pltpu.VMEM((2, WWIN, EDIM), jnp.float32),
            pltpu.VMEM((2, WWIN), jnp.float32),
            pltpu.VMEM((16,), jnp.float32),
            pltpu.VMEM_SHARED((NPAD, EDIM), jnp.float32),
            pltpu.VMEM_SHARED((NPAD,), jnp.float32),
            pltpu.SemaphoreType.DMA,
            pltpu.SemaphoreType.DMA,
            pltpu.SemaphoreType.DMA,
            pltpu.SemaphoreType.DMA,
            pltpu.SemaphoreType.DMA,
        ],
    )
    def k(ai_h, aj_h, ae_h, src_h, dst3_h, attr_h, sv_h,
          aggr_o, ssum_o,
          ai_t, aj_t, ae_c, src_c, idx_c, attr_b, wr_b, ex_b, sv_b,
          aggr_sh, s_sh, sem_in, sem_a0, sem_a1, sem_s0, sem_s1):
        c = lax.axis_index("c")
        s = lax.axis_index("s")
        wid = s * 2 + c
        ebase = wid * EPW
        sem_a = (sem_a0, sem_a1)
        sem_s = (sem_s0, sem_s1)

        d1 = pltpu.async_copy(ai_h, ai_t, sem_in)
        d2 = pltpu.async_copy(aj_h, aj_t, sem_in)
        d3 = pltpu.async_copy(ae_h.at[pl.ds(ebase, EPW)], ae_c, sem_in)
        d4 = pltpu.async_copy(src_h.at[pl.ds(ebase, EPW)], src_c, sem_in)
        d5 = pltpu.async_copy(dst3_h.at[pl.ds(wid * (EPW // 128), EPW // 128), :],
                              idx_c, sem_in)
        d6 = pltpu.async_copy(sv_h, sv_b, sem_in)

        _TAIL = E % WWIN

        def start_attr(wi, p):
            start = ebase + wi * WWIN

            @pl.when(start + WWIN <= E)
            def _():
                pltpu.async_copy(attr_h.at[pl.ds(start, WWIN), :],
                                 attr_b.at[p], sem_a[p])

            @pl.when(start + WWIN > E)
            def _():
                @pl.when(start < E)
                def _():
                    pltpu.async_copy(attr_h.at[pl.ds(E - _TAIL, _TAIL), :],
                                     attr_b.at[p, pl.ds(0, _TAIL)], sem_a[p])

        def wait_attr(wi, p):
            start = ebase + wi * WWIN

            @pl.when(start + WWIN <= E)
            def _():
                pltpu.make_async_copy(attr_h.at[pl.ds(0, WWIN), :],
                                      attr_b.at[p], sem_a[p]).wait()

            @pl.when(start + WWIN > E)
            def _():
                @pl.when(start < E)
                def _():
                    pltpu.make_async_copy(attr_h.at[pl.ds(0, _TAIL), :],
                                          attr_b.at[p, pl.ds(0, _TAIL)],
                                          sem_a[p]).wait()

        start_attr(0, 0)

        zrow = jnp.zeros((16,), jnp.float32)
        rows_per_sub = NPAD // 16

        def zfill(i, carry):
            wr_b[0, i, :] = zrow
            return carry

        lax.fori_loop(0, 128, zfill, 0)

        def zfill1(i, carry):
            ex_b[0, pl.ds(i * 16, 16)] = zrow
            return carry

        lax.fori_loop(0, rows_per_sub // 16, zfill1, 0)

        for jz in range(rows_per_sub // 128):
            pltpu.sync_copy(wr_b.at[0, pl.ds(0, 128), :],
                            aggr_sh.at[pl.ds(s * rows_per_sub + jz * 128, 128), :])
        pltpu.sync_copy(ex_b.at[0, pl.ds(0, rows_per_sub)],
                        s_sh.at[pl.ds(s * rows_per_sub, rows_per_sub)])
        for d in (d1, d2, d3, d4, d5, d6):
            d.wait()
        plsc.subcore_barrier()

        shift = sv_b[...]

        def drain_scatters(wi_prev, p):
            for jj in range(WWIN // 128):
                irow = idx_c.at[wi_prev * (WWIN // 128) + jj]
                pltpu.make_async_copy(wr_b.at[p, pl.ds(jj * 128, 128), :],
                                      aggr_sh.at[irow], sem_s[p]).wait()
                pltpu.make_async_copy(ex_b.at[p, pl.ds(jj * 128, 128)],
                                      s_sh.at[irow], sem_s[p]).wait()

        def do_window(wi, p):
            wb = wi * WWIN
            wait_attr(wi, p)

            @pl.when(wi + 1 < NWIN)
            def _():
                start_attr(wi + 1, 1 - p)

            @pl.when(wi >= 2)
            def _():
                drain_scatters(wi - 2, p)

            def group(g, gc):
                fe = wb + g * 16
                row = fe // 128
                col = fe - row * 128
                dv = idx_c[row, pl.ds(col, 16)]
                srcv = src_c[pl.ds(fe, 16)]
                aev = ae_c[pl.ds(fe, 16)]
                aiv = plsc.load_gather(ai_t, [dv])
                ajv = plsc.load_gather(aj_t, [srcv])
                pre = aiv + ajv + aev
                alpha = jnp.maximum(pre, 0.2 * pre)
                ex = jnp.exp(alpha - shift)
                eb = g * 16
                ex_b[p, pl.ds(eb, 16)] = ex
                for j in range(16):
                    wj = lax.broadcast_in_dim(ex[j], (16,), ())
                    wr_b[p, eb + j, :] = attr_b[p, eb + j, :] * wj
                return gc

            lax.fori_loop(0, WWIN // 16, group, 0)

            for jj in range(WWIN // 128):
                irow = idx_c.at[wi * (WWIN // 128) + jj]
                pltpu.async_copy(wr_b.at[p, pl.ds(jj * 128, 128), :],
                                 aggr_sh.at[irow], sem_s[p], add=True)
                pltpu.async_copy(ex_b.at[p, pl.ds(jj * 128, 128)],
                                 s_sh.at[irow], sem_s[p], add=True)

        def window2(w2, carry):
            do_window(w2 * 2, 0)
            do_window(w2 * 2 + 1, 1)
            return carry

        lax.fori_loop(0, NWIN // 2, window2, 0)
        drain_scatters(NWIN - 2, 0)
        drain_scatters(NWIN - 1, 1)
        plsc.subcore_barrier()

        @pl.when(s == 0)
        def _():
            pltpu.sync_copy(aggr_sh, aggr_o.at[c])
            pltpu.sync_copy(s_sh, ssum_o.at[c])

    return k(ai, aj, ae, src, dst3, attr_p, sv)


def _step_kernel(aggr_ref, s_ref, h_ref, h0_ref, wews_ref, bs_ref,
                 wih_ref, whh_ref, bih_ref, bhh_ref, lng_ref, lnb_ref,
                 wn_ref, wi_ref, wj_ref,
                 h_out, ai_out, aj_out, st_ref, sum_out, sm):
    i = pl.program_id(0)
    ag = aggr_ref[0] + aggr_ref[1]
    sden = s_ref[0] + s_ref[1]
    mlin = jnp.dot(ag, wews_ref[...], preferred_element_type=jnp.float32)
    mlin = mlin / (sden[:, None] + 1e-16) + bs_ref[...][None, :]
    m = _celu(mlin)

    h = h_ref[...]
    gi = lax.dot_general(m, wih_ref[...], (((1,), (1,)), ((), ())),
                         preferred_element_type=jnp.float32) + bih_ref[...][None, :]
    gh = lax.dot_general(h, whh_ref[...], (((1,), (1,)), ((), ())),
                         preferred_element_type=jnp.float32) + bhh_ref[...][None, :]
    r = jax.nn.sigmoid(gi[:, :DIM] + gh[:, :DIM])
    z = jax.nn.sigmoid(gi[:, DIM:2 * DIM] + gh[:, DIM:2 * DIM])
    n = jnp.tanh(gi[:, 2 * DIM:] + r * gh[:, 2 * DIM:])
    hn = (1.0 - z) * n + z * h
    h_out[...] = hn

    mu = jnp.mean(hn, axis=1, keepdims=True)
    var = jnp.mean((hn - mu) ** 2, axis=1, keepdims=True)
    xb = (hn - mu) / jnp.sqrt(var + 1e-5) * lng_ref[...][None, :] + lnb_ref[...][None, :]
    sum_out[...] = h0_ref[...] + xb

    xp = jnp.dot(xb, wn_ref[...], preferred_element_type=jnp.float32)
    ai = jnp.sum(xp * wi_ref[...][None, :], axis=1)
    aj = jnp.sum(xp * wj_ref[...][None, :], axis=1)
    ai_out[...] = ai
    aj_out[...] = aj

    @pl.when(i == 0)
    def _():
        sm[0] = jnp.float32(_NEG)
        sm[1] = jnp.float32(_NEG)

    sm[0] = jnp.maximum(sm[0], jnp.max(ai))
    sm[1] = jnp.maximum(sm[1], jnp.max(aj))

    @pl.when(i == NGRID - 1)
    def _():
        row = lax.broadcasted_iota(jnp.int32, (8, 128), 0)
        st_ref[...] = jnp.where(row == 0, sm[0], sm[1])


def _step(aggr_p, s_p, h, h0, WeWs, bs, W_ih, W_hh, b_ih, b_hh, ln_g, ln_b,
          Wn, wta_i, wta_j):
    return pl.pallas_call(
        _step_kernel,
        grid=(NGRID,),
        in_specs=[
            pl.BlockSpec((2, NBLK, EDIM), lambda i: (0, i, 0)),
            pl.BlockSpec((2, NBLK), lambda i: (0, i)),
            pl.BlockSpec((NBLK, DIM), lambda i: (i, 0)),
            pl.BlockSpec((NBLK, DIM), lambda i: (i, 0)),
            pl.BlockSpec((EDIM, DIM), lambda i: (0, 0)),
            pl.BlockSpec((DIM,), lambda i: (0,)),
            pl.BlockSpec((3 * DIM, DIM), lambda i: (0, 0)),
            pl.BlockSpec((3 * DIM, DIM), lambda i: (0, 0)),
            pl.BlockSpec((3 * DIM,), lambda i: (0,)),
            pl.BlockSpec((3 * DIM,), lambda i: (0,)),
            pl.BlockSpec((DIM,), lambda i: (0,)),
            pl.BlockSpec((DIM,), lambda i: (0,)),
            pl.BlockSpec((DIM, DIM), lambda i: (0, 0)),
            pl.BlockSpec((DIM,), lambda i: (0,)),
            pl.BlockSpec((DIM,), lambda i: (0,)),
        ],
        out_specs=[
            pl.BlockSpec((NBLK, DIM), lambda i: (i, 0)),
            pl.BlockSpec((NBLK,), lambda i: (i,)),
            pl.BlockSpec((NBLK,), lambda i: (i,)),
            pl.BlockSpec((8, 128), lambda i: (0, 0)),
            pl.BlockSpec((NBLK, DIM), lambda i: (i, 0)),
        ],
        out_shape=[
            jax.ShapeDtypeStruct((NPAD, DIM), jnp.float32),
            jax.ShapeDtypeStruct((NPAD,), jnp.float32),
            jax.ShapeDtypeStruct((NPAD,), jnp.float32),
            jax.ShapeDtypeStruct((8, 128), jnp.float32),
            jax.ShapeDtypeStruct((NPAD, DIM), jnp.float32),
        ],
        scratch_shapes=[pltpu.SMEM((2,), jnp.float32)],
    )(aggr_p, s_p, h, h0, WeWs, bs, W_ih, W_hh, b_ih, b_hh, ln_g, ln_b,
      Wn, wta_i, wta_j)


def kernel(x, edge_index, edge_attr, W0, b0, Wn, We, Wta, Ws, bs,
           W_ih, W_hh, b_ih, b_hh, ln_g, ln_b):
    x_p = jnp.pad(x, ((0, NPAD - N), (0, 0)))
    src = jnp.pad(edge_index[0], (0, EPAD - E))
    dst = jnp.pad(edge_index[1], (0, EPAD - E))
    dst3 = dst.reshape(EPAD // 128, 128)

    wta_i = Wta[0, :DIM]
    wta_e = Wta[0, DIM:2 * DIM]
    wta_j = Wta[0, 2 * DIM:]

    ae2, aemax = _edge_const(edge_attr.T, We, wta_e)
    ae = jnp.pad(ae2, (0, EPAD - E), constant_values=_NEG)
    mae = aemax[0, 0]

    h0, ai, aj, stat, WeWs = _prologue(x_p, W0, b0, Wn, wta_i, wta_j, We, Ws)

    h = h0
    out = None
    for _ in range(3):
        shift = jnp.maximum(jnp.float32(0.0), stat[0, 0] + stat[1, 0] + mae)
        sv = jnp.full((16,), shift, jnp.float32)
        aggr_p, s_p = _sc_round(ai, aj, ae, src, dst3, edge_attr, sv)
        h, ai, aj, stat, out = _step(aggr_p, s_p, h, h0, WeWs, bs,
                                     W_ih, W_hh, b_ih, b_hh, ln_g, ln_b,
                                     Wn, wta_i, wta_j)
    return out[:N, :]


# speedup vs baseline: 1.0282x; 1.0282x over previous
import functools

import jax
import jax.numpy as jnp
from jax import lax
from jax.experimental import pallas as pl
from jax.experimental.pallas import tpu as pltpu
from jax.experimental.pallas import tpu_sc as plsc

N = 10000
E = 320000
IN_DIM = 128
DIM = 64
EDIM = 16

NPAD = 10240
EPAD = 327680
NW = 32
EPW = EPAD // NW
WWIN = 1024
NWIN = EPW // WWIN

NBLK = 2048
NGRID = NPAD // NBLK
EBLK = 8192
EGRID = (E + EBLK - 1) // EBLK

_NEG = -1e30


def _celu(v):
    return jnp.where(v > 0, v, jnp.exp(jnp.minimum(v, 0.0)) - 1.0)


def _edge_const_kernel(attrT_ref, we_ref, wte_ref, ae_ref, mx_ref, mxs):
    i = pl.program_id(0)
    wvec = jnp.dot(we_ref[...], wte_ref[...], preferred_element_type=jnp.float32)
    a = attrT_ref[...]
    col = i * EBLK + lax.broadcasted_iota(jnp.int32, (EDIM, EBLK), 1)
    prod = jnp.where(col < E, a * wvec[:, None], 0.0)
    ae = jnp.sum(prod, axis=0)
    ae_ref[...] = ae

    @pl.when(i == 0)
    def _():
        mxs[0] = jnp.float32(_NEG)

    aemask = jnp.where(col[0] < E, ae, _NEG)
    mxs[0] = jnp.maximum(mxs[0], jnp.max(aemask))
    mx_ref[...] = jnp.full((8, 128), mxs[0], jnp.float32)


def _edge_const(attrT, We, wta_e):
    return pl.pallas_call(
        _edge_const_kernel,
        grid=(EGRID,),
        in_specs=[
            pl.BlockSpec((EDIM, EBLK), lambda i: (0, i)),
            pl.BlockSpec((EDIM, DIM), lambda i: (0, 0)),
            pl.BlockSpec((DIM,), lambda i: (0,)),
        ],
        out_specs=[
            pl.BlockSpec((EBLK,), lambda i: (i,)),
            pl.BlockSpec((8, 128), lambda i: (0, 0)),
        ],
        out_shape=[
            jax.ShapeDtypeStruct((E,), jnp.float32),
            jax.ShapeDtypeStruct((8, 128), jnp.float32),
        ],
        scratch_shapes=[pltpu.SMEM((1,), jnp.float32)],
    )(attrT, We, wta_e)


def _pre_kernel(x_ref, w0_ref, b0_ref, wn_ref, wi_ref, wj_ref, we_ref, ws_ref,
                h0_ref, ai_ref, aj_ref, st_ref, wews_ref, sm):
    i = pl.program_id(0)
    h0 = _celu(jnp.dot(x_ref[...], w0_ref[...],
                       preferred_element_type=jnp.float32) + b0_ref[...][None, :])
    h0_ref[...] = h0
    xp = jnp.dot(h0, wn_ref[...], preferred_element_type=jnp.float32)
    ai = jnp.sum(xp * wi_ref[...][None, :], axis=1)
    aj = jnp.sum(xp * wj_ref[...][None, :], axis=1)
    ai_ref[...] = ai
    aj_ref[...] = aj

    @pl.when(i == 0)
    def _():
        sm[0] = jnp.float32(_NEG)
        sm[1] = jnp.float32(_NEG)
        wews_ref[...] = jnp.dot(we_ref[...], ws_ref[...],
                                preferred_element_type=jnp.float32)

    sm[0] = jnp.maximum(sm[0], jnp.max(ai))
    sm[1] = jnp.maximum(sm[1], jnp.max(aj))

    @pl.when(i == NGRID - 1)
    def _():
        row = lax.broadcasted_iota(jnp.int32, (8, 128), 0)
        st_ref[...] = jnp.where(row == 0, sm[0], sm[1])


def _prologue(x_p, W0, b0, Wn, wta_i, wta_j, We, Ws):
    return pl.pallas_call(
        _pre_kernel,
        grid=(NGRID,),
        in_specs=[
            pl.BlockSpec((NBLK, IN_DIM), lambda i: (i, 0)),
            pl.BlockSpec((IN_DIM, DIM), lambda i: (0, 0)),
            pl.BlockSpec((DIM,), lambda i: (0,)),
            pl.BlockSpec((DIM, DIM), lambda i: (0, 0)),
            pl.BlockSpec((DIM,), lambda i: (0,)),
            pl.BlockSpec((DIM,), lambda i: (0,)),
            pl.BlockSpec((EDIM, DIM), lambda i: (0, 0)),
            pl.BlockSpec((DIM, DIM), lambda i: (0, 0)),
        ],
        out_specs=[
            pl.BlockSpec((NBLK, DIM), lambda i: (i, 0)),
            pl.BlockSpec((NBLK,), lambda i: (i,)),
            pl.BlockSpec((NBLK,), lambda i: (i,)),
            pl.BlockSpec((8, 128), lambda i: (0, 0)),
            pl.BlockSpec((EDIM, DIM), lambda i: (0, 0)),
        ],
        out_shape=[
            jax.ShapeDtypeStruct((NPAD, DIM), jnp.float32),
            jax.ShapeDtypeStruct((NPAD,), jnp.float32),
            jax.ShapeDtypeStruct((NPAD,), jnp.float32),
            jax.ShapeDtypeStruct((8, 128), jnp.float32),
            jax.ShapeDtypeStruct((EDIM, DIM), jnp.float32),
        ],
        scratch_shapes=[pltpu.SMEM((2,), jnp.float32)],
    )(x_p, W0, b0, Wn, wta_i, wta_j, We, Ws)


def _sc_round(ai, aj, ae, src, dst3, attr_p, sv):
    mesh = plsc.VectorSubcoreMesh(core_axis_name="c", subcore_axis_name="s")

    @functools.partial(
        pl.kernel,
        out_type=[
            jax.ShapeDtypeStruct((2, NPAD, EDIM), jnp.float32),
            jax.ShapeDtypeStruct((2, NPAD), jnp.float32),
        ],
        mesh=mesh,
        compiler_params=pltpu.CompilerParams(needs_layout_passes=False,
                                             use_tc_tiling_on_sc=False),
        scratch_types=[
            pltpu.VMEM((NPAD,), jnp.float32),
            pltpu.VMEM((NPAD,), jnp.float32),
            pltpu.VMEM((EPW,), jnp.float32),
            pltpu.VMEM((EPW,), jnp.int32),
            pltpu.VMEM((EPW // 128, 128), jnp.int32),
            pltpu.VMEM((2, WWIN, EDIM), jnp.float32),
            pltpu.VMEM((2, WWIN, EDIM), jnp.float32),
            pltpu.VMEM((2, WWIN), jnp.float32),
            pltpu.VMEM((16,), jnp.float32),
            pltpu.VMEM_SHARED((NPAD, EDIM), jnp.float32),
            pltpu.VMEM_SHARED((NPAD,), jnp.float32),
            pltpu.SemaphoreType.DMA,
            pltpu.SemaphoreType.DMA,
            pltpu.SemaphoreType.DMA,
            pltpu.SemaphoreType.DMA,
            pltpu.SemaphoreType.DMA,
        ],
    )
    def k(ai_h, aj_h, ae_h, src_h, dst3_h, attr_h, sv_h,
          aggr_o, ssum_o,
          ai_t, aj_t, ae_c, src_c, idx_c, attr_b, wr_b, ex_b, sv_b,
          aggr_sh, s_sh, sem_in, sem_a0, sem_a1, sem_s0, sem_s1):
        c = lax.axis_index("c")
        s = lax.axis_index("s")
        wid = s * 2 + c
        ebase = wid * EPW
        sem_a = (sem_a0, sem_a1)
        sem_s = (sem_s0, sem_s1)

        d1 = pltpu.async_copy(ai_h, ai_t, sem_in)
        d2 = pltpu.async_copy(aj_h, aj_t, sem_in)
        d3 = pltpu.async_copy(ae_h.at[pl.ds(ebase, EPW)], ae_c, sem_in)
        d4 = pltpu.async_copy(src_h.at[pl.ds(ebase, EPW)], src_c, sem_in)
        d5 = pltpu.async_copy(dst3_h.at[pl.ds(wid * (EPW // 128), EPW // 128), :],
                              idx_c, sem_in)
        d6 = pltpu.async_copy(sv_h, sv_b, sem_in)

        _TAIL = E % WWIN

        def start_attr(wi, p):
            start = ebase + wi * WWIN

            @pl.when(start + WWIN <= E)
            def _():
                pltpu.async_copy(attr_h.at[pl.ds(start, WWIN), :],
                                 attr_b.at[p], sem_a[p])

            @pl.when(start + WWIN > E)
            def _():
                @pl.when(start < E)
                def _():
                    pltpu.async_copy(attr_h.at[pl.ds(E - _TAIL, _TAIL), :],
                                     attr_b.at[p, pl.ds(0, _TAIL)], sem_a[p])

        def wait_attr(wi, p):
            start = ebase + wi * WWIN

            @pl.when(start + WWIN <= E)
            def _():
                pltpu.make_async_copy(attr_h.at[pl.ds(0, WWIN), :],
                                      attr_b.at[p], sem_a[p]).wait()

            @pl.when(start + WWIN > E)
            def _():
                @pl.when(start < E)
                def _():
                    pltpu.make_async_copy(attr_h.at[pl.ds(0, _TAIL), :],
                                          attr_b.at[p, pl.ds(0, _TAIL)],
                                          sem_a[p]).wait()

        start_attr(0, 0)

        zrow = jnp.zeros((16,), jnp.float32)
        rows_per_sub = NPAD // 16

        def zfill(i, carry):
            wr_b[0, i, :] = zrow
            return carry

        lax.fori_loop(0, 128, zfill, 0)

        def zfill1(i, carry):
            ex_b[0, pl.ds(i * 16, 16)] = zrow
            return carry

        lax.fori_loop(0, rows_per_sub // 16, zfill1, 0)

        for jz in range(rows_per_sub // 128):
            pltpu.sync_copy(wr_b.at[0, pl.ds(0, 128), :],
                            aggr_sh.at[pl.ds(s * rows_per_sub + jz * 128, 128), :])
        pltpu.sync_copy(ex_b.at[0, pl.ds(0, rows_per_sub)],
                        s_sh.at[pl.ds(s * rows_per_sub, rows_per_sub)])
        for d in (d1, d2, d3, d4, d5, d6):
            d.wait()
        plsc.subcore_barrier()

        shift = sv_b[...]

        def drain_scatters(wi_prev, p):
            for jj in range(WWIN // 128):
                irow = idx_c.at[wi_prev * (WWIN // 128) + jj]
                pltpu.make_async_copy(wr_b.at[p, pl.ds(jj * 128, 128), :],
                                      aggr_sh.at[irow], sem_s[p]).wait()
                pltpu.make_async_copy(ex_b.at[p, pl.ds(jj * 128, 128)],
                                      s_sh.at[irow], sem_s[p]).wait()

        def do_window(wi, p):
            wb = wi * WWIN
            wait_attr(wi, p)

            @pl.when(wi + 1 < NWIN)
            def _():
                start_attr(wi + 1, 1 - p)

            @pl.when(wi >= 2)
            def _():
                drain_scatters(wi - 2, p)

            def group(g, gc):
                fe0 = wb + g * 32
                row = fe0 // 128
                col = fe0 - row * 128
                exs = []
                for h in range(2):
                    fe = fe0 + h * 16
                    dv = idx_c[row, pl.ds(col + h * 16, 16)]
                    srcv = src_c[pl.ds(fe, 16)]
                    aev = ae_c[pl.ds(fe, 16)]
                    aiv = plsc.load_gather(ai_t, [dv])
                    ajv = plsc.load_gather(aj_t, [srcv])
                    pre = aiv + ajv + aev
                    alpha = jnp.maximum(pre, 0.2 * pre)
                    exs.append(jnp.exp(alpha - shift))
                eb = g * 32
                ex_b[p, pl.ds(eb, 16)] = exs[0]
                ex_b[p, pl.ds(eb + 16, 16)] = exs[1]
                for h in range(2):
                    for j in range(16):
                        wj = lax.broadcast_in_dim(exs[h][j], (16,), ())
                        wr_b[p, eb + h * 16 + j, :] = attr_b[p, eb + h * 16 + j, :] * wj
                return gc

            lax.fori_loop(0, WWIN // 32, group, 0)

            for jj in range(WWIN // 128):
                irow = idx_c.at[wi * (WWIN // 128) + jj]
                pltpu.async_copy(wr_b.at[p, pl.ds(jj * 128, 128), :],
                                 aggr_sh.at[irow], sem_s[p], add=True)
                pltpu.async_copy(ex_b.at[p, pl.ds(jj * 128, 128)],
                                 s_sh.at[irow], sem_s[p], add=True)

        def window2(w2, carry):
            do_window(w2 * 2, 0)
            do_window(w2 * 2 + 1, 1)
            return carry

        lax.fori_loop(0, NWIN // 2, window2, 0)
        drain_scatters(NWIN - 2, 0)
        drain_scatters(NWIN - 1, 1)
        plsc.subcore_barrier()

        @pl.when(s == 0)
        def _():
            pltpu.sync_copy(aggr_sh, aggr_o.at[c])
            pltpu.sync_copy(s_sh, ssum_o.at[c])

    return k(ai, aj, ae, src, dst3, attr_p, sv)


def _step_kernel(aggr_ref, s_ref, h_ref, h0_ref, wews_ref, bs_ref,
                 wih_ref, whh_ref, bih_ref, bhh_ref, lng_ref, lnb_ref,
                 wn_ref, wi_ref, wj_ref,
                 h_out, ai_out, aj_out, st_ref, sum_out, sm):
    i = pl.program_id(0)
    ag = aggr_ref[0] + aggr_ref[1]
    sden = s_ref[0] + s_ref[1]
    mlin = jnp.dot(ag, wews_ref[...], preferred_element_type=jnp.float32)
    mlin = mlin / (sden[:, None] + 1e-16) + bs_ref[...][None, :]
    m = _celu(mlin)

    h = h_ref[...]
    gi = lax.dot_general(m, wih_ref[...], (((1,), (1,)), ((), ())),
                         preferred_element_type=jnp.float32) + bih_ref[...][None, :]
    gh = lax.dot_general(h, whh_ref[...], (((1,), (1,)), ((), ())),
                         preferred_element_type=jnp.float32) + bhh_ref[...][None, :]
    r = jax.nn.sigmoid(gi[:, :DIM] + gh[:, :DIM])
    z = jax.nn.sigmoid(gi[:, DIM:2 * DIM] + gh[:, DIM:2 * DIM])
    n = jnp.tanh(gi[:, 2 * DIM:] + r * gh[:, 2 * DIM:])
    hn = (1.0 - z) * n + z * h
    h_out[...] = hn

    mu = jnp.mean(hn, axis=1, keepdims=True)
    var = jnp.mean((hn - mu) ** 2, axis=1, keepdims=True)
    xb = (hn - mu) / jnp.sqrt(var + 1e-5) * lng_ref[...][None, :] + lnb_ref[...][None, :]
    sum_out[...] = h0_ref[...] + xb

    xp = jnp.dot(xb, wn_ref[...], preferred_element_type=jnp.float32)
    ai = jnp.sum(xp * wi_ref[...][None, :], axis=1)
    aj = jnp.sum(xp * wj_ref[...][None, :], axis=1)
    ai_out[...] = ai
    aj_out[...] = aj

    @pl.when(i == 0)
    def _():
        sm[0] = jnp.float32(_NEG)
        sm[1] = jnp.float32(_NEG)

    sm[0] = jnp.maximum(sm[0], jnp.max(ai))
    sm[1] = jnp.maximum(sm[1], jnp.max(aj))

    @pl.when(i == NGRID - 1)
    def _():
        row = lax.broadcasted_iota(jnp.int32, (8, 128), 0)
        st_ref[...] = jnp.where(row == 0, sm[0], sm[1])


def _step(aggr_p, s_p, h, h0, WeWs, bs, W_ih, W_hh, b_ih, b_hh, ln_g, ln_b,
          Wn, wta_i, wta_j):
    return pl.pallas_call(
        _step_kernel,
        grid=(NGRID,),
        in_specs=[
            pl.BlockSpec((2, NBLK, EDIM), lambda i: (0, i, 0)),
            pl.BlockSpec((2, NBLK), lambda i: (0, i)),
            pl.BlockSpec((NBLK, DIM), lambda i: (i, 0)),
            pl.BlockSpec((NBLK, DIM), lambda i: (i, 0)),
            pl.BlockSpec((EDIM, DIM), lambda i: (0, 0)),
            pl.BlockSpec((DIM,), lambda i: (0,)),
            pl.BlockSpec((3 * DIM, DIM), lambda i: (0, 0)),
            pl.BlockSpec((3 * DIM, DIM), lambda i: (0, 0)),
            pl.BlockSpec((3 * DIM,), lambda i: (0,)),
            pl.BlockSpec((3 * DIM,), lambda i: (0,)),
            pl.BlockSpec((DIM,), lambda i: (0,)),
            pl.BlockSpec((DIM,), lambda i: (0,)),
            pl.BlockSpec((DIM, DIM), lambda i: (0, 0)),
            pl.BlockSpec((DIM,), lambda i: (0,)),
            pl.BlockSpec((DIM,), lambda i: (0,)),
        ],
        out_specs=[
            pl.BlockSpec((NBLK, DIM), lambda i: (i, 0)),
            pl.BlockSpec((NBLK,), lambda i: (i,)),
            pl.BlockSpec((NBLK,), lambda i: (i,)),
            pl.BlockSpec((8, 128), lambda i: (0, 0)),
            pl.BlockSpec((NBLK, DIM), lambda i: (i, 0)),
        ],
        out_shape=[
            jax.ShapeDtypeStruct((NPAD, DIM), jnp.float32),
            jax.ShapeDtypeStruct((NPAD,), jnp.float32),
            jax.ShapeDtypeStruct((NPAD,), jnp.float32),
            jax.ShapeDtypeStruct((8, 128), jnp.float32),
            jax.ShapeDtypeStruct((NPAD, DIM), jnp.float32),
        ],
        scratch_shapes=[pltpu.SMEM((2,), jnp.float32)],
    )(aggr_p, s_p, h, h0, WeWs, bs, W_ih, W_hh, b_ih, b_hh, ln_g, ln_b,
      Wn, wta_i, wta_j)


def kernel(x, edge_index, edge_attr, W0, b0, Wn, We, Wta, Ws, bs,
           W_ih, W_hh, b_ih, b_hh, ln_g, ln_b):
    x_p = jnp.pad(x, ((0, NPAD - N), (0, 0)))
    src = jnp.pad(edge_index[0], (0, EPAD - E))
    dst = jnp.pad(edge_index[1], (0, EPAD - E))
    dst3 = dst.reshape(EPAD // 128, 128)

    wta_i = Wta[0, :DIM]
    wta_e = Wta[0, DIM:2 * DIM]
    wta_j = Wta[0, 2 * DIM:]

    ae2, aemax = _edge_const(edge_attr.T, We, wta_e)
    ae = jnp.pad(ae2, (0, EPAD - E), constant_values=_NEG)
    mae = aemax[0, 0]

    h0, ai, aj, stat, WeWs = _prologue(x_p, W0, b0, Wn, wta_i, wta_j, We, Ws)

    h = h0
    out = None
    for _ in range(3):
        shift = jnp.maximum(jnp.float32(0.0), stat[0, 0] + stat[1, 0] + mae)
        sv = jnp.full((16,), shift, jnp.float32)
        aggr_p, s_p = _sc_round(ai, aj, ae, src, dst3, edge_attr, sv)
        h, ai, aj, stat, out = _step(aggr_p, s_p, h, h0, WeWs, bs,
                                     W_ih, W_hh, b_ih, b_hh, ln_g, ln_b,
                                     Wn, wta_i, wta_j)
    return out[:N, :]
